# Initial kernel scaffold; baseline (speedup 1.0000x reference)
import jax
import jax.numpy as jnp
from jax.experimental import pallas as pl


def kernel(prefix_indices, suffix_indices, prefix_table, suffix_table):
    raise NotImplementedError("write your pallas kernel here")


# speedup vs baseline: 3.5878x; 3.5878x over previous
import functools

import jax
import jax.numpy as jnp
from jax import lax
from jax.experimental import pallas as pl
from jax.experimental.pallas import tpu as pltpu
from jax.experimental.pallas import tpu_sc as plsc

NUM_PREFIXES = 100000
EMBED_DIM = 16

NC = 2
NSC = 16
NW = NC * NSC

IDX_PER_DMA = 128
K = 20
T = K * IDX_PER_DMA
S = 20


def _sc_gather(table, idx):
  tot = NW * S * T
  mesh = plsc.VectorSubcoreMesh(core_axis_name="c", subcore_axis_name="s")

  @functools.partial(
      pl.kernel,
      out_type=jax.ShapeDtypeStruct((tot, EMBED_DIM), jnp.float32),
      mesh=mesh,
      scratch_types=[
          pltpu.VMEM((K, IDX_PER_DMA), jnp.int32),
          pltpu.VMEM((T, EMBED_DIM), jnp.float32),
          pltpu.SemaphoreType.DMA,
      ],
      compiler_params=pltpu.CompilerParams(use_tc_tiling_on_sc=False),
  )
  def k(table_hbm, idx_hbm, out_hbm, idx_v, rows_v, sem):
    wid = lax.axis_index("s") * NC + lax.axis_index("c")

    def step(s, carry):
      blk = wid * S + s
      pltpu.sync_copy(idx_hbm.at[blk], idx_v)
      cps = [
          pltpu.async_copy(
              table_hbm.at[idx_v.at[j]],
              rows_v.at[pl.ds(j * IDX_PER_DMA, IDX_PER_DMA)],
              sem,
          )
          for j in range(K)
      ]
      for c in cps:
        c.wait()
      pltpu.sync_copy(rows_v, out_hbm.at[pl.ds(blk * T, T)])
      return carry

    lax.fori_loop(0, S, step, 0)

  return k(table, idx)


def kernel(prefix_indices, suffix_indices, prefix_table, suffix_table):
  batch, hist = prefix_indices.shape
  pidx = prefix_indices.reshape(-1).astype(jnp.int32)
  sidx = (suffix_indices.reshape(-1) + NUM_PREFIXES).astype(jnp.int32)
  comb = jnp.stack([pidx, sidx], axis=1).reshape(NW * S, K, IDX_PER_DMA)
  cat = jnp.concatenate([prefix_table, suffix_table], axis=0)
  out2 = _sc_gather(cat, comb)
  return out2.reshape(batch, hist, 2 * EMBED_DIM)


# speedup vs baseline: 4.7055x; 1.3115x over previous
import functools

import jax
import jax.numpy as jnp
from jax import lax
from jax.experimental import pallas as pl
from jax.experimental.pallas import tpu as pltpu
from jax.experimental.pallas import tpu_sc as plsc

VOCAB = 100000
EMBED_DIM = 16

NC = 2
NSC = 16

IDX_PER_DMA = 128
K = 10
T = K * IDX_PER_DMA
S = 40
ROWS_PER_TEC_LOAD = VOCAB // NSC


def _sc_dual_gather(tables, idx):
  n = NSC * S * T
  mesh = plsc.VectorSubcoreMesh(core_axis_name="c", subcore_axis_name="s")

  @functools.partial(
      pl.kernel,
      out_type=jax.ShapeDtypeStruct((n, 2 * EMBED_DIM), jnp.float32),
      mesh=mesh,
      scratch_types=[
          pltpu.VMEM_SHARED((VOCAB, EMBED_DIM), jnp.float32),
          pltpu.VMEM((K, IDX_PER_DMA), jnp.int32),
          pltpu.VMEM((T, EMBED_DIM), jnp.float32),
          pltpu.SemaphoreType.DMA,
      ],
      compiler_params=pltpu.CompilerParams(use_tc_tiling_on_sc=False),
  )
  def k(tables_hbm, idx_hbm, out_hbm, table_sh, idx_v, rows_v, sem):
    cid = lax.axis_index("c")
    tid = lax.axis_index("s")

    slab = tid * ROWS_PER_TEC_LOAD
    pltpu.sync_copy(
        tables_hbm.at[pl.ds(cid * VOCAB + slab, ROWS_PER_TEC_LOAD)],
        table_sh.at[pl.ds(slab, ROWS_PER_TEC_LOAD)],
    )
    plsc.subcore_barrier()

    def step(s, carry):
      blk = tid * S + s
      pltpu.sync_copy(idx_hbm.at[cid, blk], idx_v)
      cps = [
          pltpu.async_copy(
              table_sh.at[idx_v.at[j]],
              rows_v.at[pl.ds(j * IDX_PER_DMA, IDX_PER_DMA)],
              sem,
          )
          for j in range(K)
      ]
      for c in cps:
        c.wait()
      pltpu.sync_copy(
          rows_v,
          out_hbm.at[pl.ds(blk * T, T), pl.ds(cid * EMBED_DIM, EMBED_DIM)],
      )
      return carry

    lax.fori_loop(0, S, step, 0)

  return k(tables, idx)


def kernel(prefix_indices, suffix_indices, prefix_table, suffix_table):
  batch, hist = prefix_indices.shape
  pidx = prefix_indices.reshape(-1).astype(jnp.int32)
  sidx = suffix_indices.reshape(-1).astype(jnp.int32)
  comb = jnp.stack([pidx, sidx]).reshape(NC, NSC * S, K, IDX_PER_DMA)
  cat = jnp.concatenate([prefix_table, suffix_table], axis=0)
  out = _sc_dual_gather(cat, comb)
  return out.reshape(batch, hist, 2 * EMBED_DIM)
